# speedup vs baseline: 1.0597x; 1.0597x over previous
import functools

import jax
import jax.numpy as jnp
from jax.experimental import pallas as pl

NUM_EXPERTS = 64
K = 8
NUM_SHARED = 2
K_SELECT = K - NUM_SHARED
END_IDX = NUM_EXPERTS - NUM_SHARED

BLOCK_ROWS = 2048


def _topk_body(gates_ref, idx_ref, gate_ref):
    x = gates_ref[...]
    b = x.shape[0]
    ns = x[:, :END_IDX]
    col = jax.lax.broadcasted_iota(jnp.int32, (b, END_IDX), 1)
    neg_inf = jnp.float32(-jnp.inf)

    vals = []
    idxs = []
    work = ns
    for _ in range(K_SELECT):
        m = jnp.max(work, axis=1)
        cand = jnp.where(work == m[:, None], col, jnp.int32(END_IDX))
        i = jnp.min(cand, axis=1).astype(jnp.int32)
        vals.append(m)
        idxs.append(i)
        work = jnp.where(col == i[:, None], neg_inf, work)

    shared_vals = [x[:, END_IDX + j] for j in range(NUM_SHARED)]
    shared_idx = [jnp.full((b,), END_IDX + j, dtype=jnp.int32)
                  for j in range(NUM_SHARED)]

    sel = jnp.stack(vals + shared_vals, axis=1)
    sel_idx = jnp.stack(idxs + shared_idx, axis=1)

    m8 = jnp.max(sel, axis=1, keepdims=True)
    e = jnp.exp(sel - m8)
    sm = e / jnp.sum(e, axis=1, keepdims=True)

    idx_ref[...] = sel_idx
    gate_ref[...] = sm


@jax.jit
def kernel(gates):
    batch = gates.shape[0]
    grid = (batch // BLOCK_ROWS,)
    out_idx, out_gate = pl.pallas_call(
        _topk_body,
        grid=grid,
        in_specs=[pl.BlockSpec((BLOCK_ROWS, NUM_EXPERTS), lambda i: (i, 0))],
        out_specs=[
            pl.BlockSpec((BLOCK_ROWS, K), lambda i: (i, 0)),
            pl.BlockSpec((BLOCK_ROWS, K), lambda i: (i, 0)),
        ],
        out_shape=[
            jax.ShapeDtypeStruct((batch, K), jnp.int32),
            jax.ShapeDtypeStruct((batch, K), jnp.float32),
        ],
    )(gates)
    return out_idx, out_gate
